# baseline (speedup 1.0000x reference)
import dataclasses
import functools

import jax
import jax.numpy as jnp
from jax import lax
from jax.experimental import pallas as pl
from jax.experimental.pallas import tpu as pltpu
from jax.experimental.pallas import tpu_sc as plsc

N_NEIGHBORS = 16
N_CLASSES = 10
TILE = 1024
CHUNK = 8
NCH = TILE // CHUNK
QB3 = 128
PAD_VAL = 1.0e9


def _extract_topk_with_labels(vals, labs, k, want_vals=True):
    B, W = vals.shape
    iota = lax.broadcasted_iota(jnp.int32, (B, W), 1)
    out_v, out_l = [], []
    for _ in range(k):
        m = jnp.min(vals, axis=1, keepdims=True)
        am = jnp.min(jnp.where(vals == m, iota, W), axis=1, keepdims=True)
        sel = iota == am
        lab = jnp.min(jnp.where(sel, labs, jnp.int32(2**30)), axis=1,
                      keepdims=True)
        vals = jnp.where(sel, jnp.float32(jnp.inf), vals)
        out_v.append(m)
        out_l.append(lab)
    vcat = jnp.concatenate(out_v, axis=1) if want_vals else None
    return vcat, jnp.concatenate(out_l, axis=1)


def _vote(run_v, run_l):
    B = run_v.shape[0]
    d = jnp.sqrt(jnp.maximum(run_v, 0.0))
    dinv = 1.0 / d
    inf_mask = jnp.isinf(dinv)
    inf_row = jnp.max(inf_mask.astype(jnp.float32), axis=1, keepdims=True) > 0
    w = jnp.where(inf_row, inf_mask.astype(jnp.float32), dinv)
    votes = [
        jnp.sum(jnp.where(run_l == c, w, 0.0), axis=1, keepdims=True)
        for c in range(N_CLASSES)
    ]
    proba = jnp.concatenate(votes, axis=1)
    s = jnp.sum(proba, axis=1, keepdims=True)
    s = jnp.where(s == 0.0, jnp.float32(1.0), s)
    proba = proba * (1.0 / s)
    pm = jnp.max(proba, axis=1, keepdims=True)
    ci = lax.broadcasted_iota(jnp.int32, (B, N_CLASSES), 1)
    pred = jnp.min(jnp.where(proba == pm, ci, jnp.int32(N_CLASSES)), axis=1,
                   keepdims=True)
    return pred, proba


def _stage_a_kernel(x_ref, t_ref, m_ref, tm_ref):
    x = x_ref[...]
    t = t_ref[0]
    dn = (((1,), (1,)), ((), ()))
    xm2 = -2.0 * x
    xym2 = lax.dot_general(xm2, t, dn, preferred_element_type=jnp.float32)
    ones = jnp.ones((8, x.shape[1]), jnp.float32)
    t2r = lax.dot_general(ones, t * t, dn,
                          preferred_element_type=jnp.float32,
                          precision=lax.Precision.HIGHEST)[0:1]
    s = t2r + xym2
    f1 = jnp.minimum(s[:, :512], s[:, 512:])
    f2 = jnp.minimum(f1[:, :256], f1[:, 256:])
    mt = jnp.minimum(f2[:, :128], f2[:, 128:])
    m_ref[0] = mt
    tm_ref[0] = jnp.min(mt, axis=1, keepdims=True)


def _stage_a(x, tr3, n_tiles):
    B, F = x.shape
    return pl.pallas_call(
        _stage_a_kernel,
        grid=(n_tiles,),
        in_specs=[
            pl.BlockSpec((B, F), lambda i: (0, 0)),
            pl.BlockSpec((1, TILE, F), lambda i: (i, 0, 0)),
        ],
        out_specs=[
            pl.BlockSpec((1, B, NCH), lambda i: (i, 0, 0)),
            pl.BlockSpec((1, B, 1), lambda i: (i, 0, 0)),
        ],
        out_shape=[
            jax.ShapeDtypeStruct((n_tiles, B, NCH), jnp.float32),
            jax.ShapeDtypeStruct((n_tiles, B, 1), jnp.float32),
        ],
    )(x, tr3)


def _stage_b_kernel(b_total, m2_ref, gidx_ref, sig_ref):
    m2t = m2_ref[...].T
    B, n_tiles = m2t.shape
    iota = lax.broadcasted_iota(jnp.int32, (B, n_tiles), 1)
    _, sig = _extract_topk_with_labels(m2t, iota, N_NEIGHBORS, want_vals=False)
    qidx = lax.broadcasted_iota(jnp.int32, (B, 1), 0)
    gidx_ref[...] = sig * b_total + qidx
    sig_ref[...] = sig


def _stage_b(m2):
    n_tiles, B = m2.shape
    return pl.pallas_call(
        functools.partial(_stage_b_kernel, B),
        in_specs=[pl.BlockSpec((n_tiles, B), lambda: (0, 0))],
        out_specs=[
            pl.BlockSpec((B, N_NEIGHBORS), lambda: (0, 0)),
            pl.BlockSpec((B, N_NEIGHBORS), lambda: (0, 0)),
        ],
        out_shape=[
            jax.ShapeDtypeStruct((B, N_NEIGHBORS), jnp.int32),
            jax.ShapeDtypeStruct((B, N_NEIGHBORS), jnp.int32),
        ],
    )(m2)


def _stage_c2_kernel(n_real, cand_ref, sig_ref, raw_ref, clamp_ref):
    sig = sig_ref[...]
    i128 = lax.broadcasted_iota(jnp.int32, (1, NCH), 1)
    kaps = []
    for k in range(N_NEIGHBORS):
        kaps.append(sig[:, k:k + 1] * NCH + i128)
    kappa = jnp.concatenate(kaps, axis=1)
    _, kap = _extract_topk_with_labels(cand_ref[...], kappa,
                                       N_NEIGHBORS, want_vals=False)
    i4 = lax.broadcasted_iota(jnp.int32, (1, CHUNK // 2), 1)
    rows = []
    prow = []
    for k in range(N_NEIGHBORS):
        kk = kap[:, k:k + 1]
        rows.append((kk >> 7) * TILE + (kk & 127) + 256 * i4)
        prow.append((kk >> 7) * (TILE // 2) + (kk & 127) + 128 * i4)
    raw0 = jnp.concatenate(rows, axis=1)
    raw_ref[...] = jnp.concatenate([raw0, raw0 + 128], axis=1)
    clamp_ref[...] = jnp.concatenate(prow, axis=1)


def _stage_c2(cand, sig, n_real):
    B = sig.shape[0]
    W = N_NEIGHBORS * CHUNK
    return pl.pallas_call(
        functools.partial(_stage_c2_kernel, n_real),
        in_specs=[
            pl.BlockSpec((B, N_NEIGHBORS * NCH), lambda: (0, 0)),
            pl.BlockSpec((B, N_NEIGHBORS), lambda: (0, 0)),
        ],
        out_specs=[
            pl.BlockSpec((B, W), lambda: (0, 0)),
            pl.BlockSpec((B, W // 2), lambda: (0, 0)),
        ],
        out_shape=[
            jax.ShapeDtypeStruct((B, W), jnp.int32),
            jax.ShapeDtypeStruct((B, W // 2), jnp.int32),
        ],
    )(cand, sig)


def _stage_c3_kernel(n_real, n_blocks, x_ref, g_ref, raw_ref, pred_ref,
                     proba_ref, d2s_ref, lab_s_ref):
    gi = pl.program_id(0)
    x = x_ref[...]
    g = g_ref[...]
    raw = raw_ref[...]
    x2 = jnp.sum(x * x, axis=1, keepdims=True)
    xb = x.astype(jnp.bfloat16).astype(jnp.float32)
    halves = []
    labs = []
    for p in range(2):
        gt = g[:, :, 64 * p:64 * (p + 1)]
        gb = gt.astype(jnp.bfloat16).astype(jnp.float32)
        xg = jnp.sum(gb * xb[:, None, :], axis=2)
        g2 = jnp.sum(gt * gt, axis=2)
        halves.append((x2 + g2) - 2.0 * xg)
        labs.append(jnp.bitwise_and(
            lax.bitcast_convert_type(g[:, :, 64 * p], jnp.int32),
            jnp.int32(15)))
    d2 = jnp.concatenate(halves, axis=1)
    d2 = jnp.where(raw >= n_real, jnp.float32(jnp.inf), d2)
    d2s_ref[pl.ds(gi * QB3, QB3), :] = d2
    lab_s_ref[pl.ds(gi * QB3, QB3), :] = jnp.concatenate(labs, axis=1)

    @pl.when(gi == n_blocks - 1)
    def _finish():
        vals, labs = _extract_topk_with_labels(d2s_ref[...], lab_s_ref[...],
                                               N_NEIGHBORS)
        pred, proba = _vote(vals, labs)
        pred_ref[...] = pred
        proba_ref[...] = proba


def _stage_c3(x, gath, raw, n_real):
    B, F = x.shape
    W = N_NEIGHBORS * CHUNK
    n_blocks = B // QB3
    return pl.pallas_call(
        functools.partial(_stage_c3_kernel, n_real, n_blocks),
        grid=(n_blocks,),
        in_specs=[
            pl.BlockSpec((QB3, F), lambda g: (g, 0)),
            pl.BlockSpec((QB3, W // 2, 128), lambda g: (g, 0, 0)),
            pl.BlockSpec((QB3, W), lambda g: (g, 0)),
        ],
        out_specs=[
            pl.BlockSpec((B, 1), lambda g: (0, 0)),
            pl.BlockSpec((B, N_CLASSES), lambda g: (0, 0)),
        ],
        out_shape=[
            jax.ShapeDtypeStruct((B, 1), jnp.int32),
            jax.ShapeDtypeStruct((B, N_CLASSES), jnp.float32),
        ],
        scratch_shapes=[
            pltpu.VMEM((B, W), jnp.float32),
            pltpu.VMEM((B, W), jnp.int32),
        ],
    )(x, gath, raw)


def _sc_gather(data, idx_flat, window=128):
    K = idx_flat.shape[0]
    D = data.shape[1]
    idx2 = idx_flat.reshape(1, K)
    mesh = plsc.VectorSubcoreMesh(core_axis_name="c", subcore_axis_name="s")
    cp = pltpu.CompilerParams()
    if "needs_layout_passes" in pltpu.CompilerParams.__dataclass_fields__:
        cp = dataclasses.replace(cp, needs_layout_passes=False)

    @functools.partial(
        pl.kernel,
        out_type=jax.ShapeDtypeStruct((K, D), data.dtype),
        mesh=mesh,
        compiler_params=cp,
    )
    def gk(x_hbm, i_hbm, o_hbm):
        def body(i_vmem, o_vmem):
            pltpu.sync_copy(x_hbm.at[i_vmem.at[0]], o_vmem)

        pltpu.emit_pipeline(
            body,
            grid=(K // window,),
            in_specs=[pl.BlockSpec((1, window), lambda i: (0, i))],
            out_specs=[pl.BlockSpec((window, D), lambda i: (i, 0))],
            core_axis_name=("c", "s"),
            dimension_semantics=(pltpu.PARALLEL,),
        )(i_hbm, o_hbm)

    return gk(data, idx2)


def kernel(x, train_data, train_labels):
    B, F = x.shape
    N = train_data.shape[0]
    n_tiles = (N + TILE - 1) // TILE
    n_pad = n_tiles * TILE
    tr3 = jnp.pad(train_data, ((0, n_pad - N), (0, 0)),
                  constant_values=PAD_VAL).reshape(n_tiles, TILE, F)
    f0i = lax.bitcast_convert_type(train_data[:, 0], jnp.int32)
    f0p = lax.bitcast_convert_type(
        jnp.bitwise_or(jnp.bitwise_and(f0i, jnp.int32(~15)),
                       train_labels.astype(jnp.int32)), jnp.float32)
    tdp = jnp.concatenate([f0p[:, None], train_data[:, 1:]], axis=1)
    trpk = jnp.pad(tdp, ((0, n_pad - N), (0, 0)), constant_values=PAD_VAL)
    ptab = trpk.reshape(n_tiles, 4, 2, 128, F).transpose(0, 1, 3, 2, 4)
    ptab = ptab.reshape(n_tiles * (TILE // 2), 2 * F)

    m3, tm = _stage_a(x, tr3, n_tiles)
    gidx, sig = _stage_b(tm.reshape(n_tiles, B))
    g1 = _sc_gather(m3.reshape(n_tiles * B, NCH),
                    gidx.reshape(B * N_NEIGHBORS))
    cand = g1.reshape(B, N_NEIGHBORS * NCH)
    raw, prow = _stage_c2(cand, sig, N)
    g2 = _sc_gather(ptab, prow.reshape(B * N_NEIGHBORS * CHUNK // 2))
    gath = g2.reshape(B, N_NEIGHBORS * CHUNK // 2, 2 * F)
    pred2, proba = _stage_c3(x, gath, raw, N)
    return (pred2.reshape(B), proba)


# speedup vs baseline: 1.0623x; 1.0623x over previous
import dataclasses
import functools

import jax
import jax.numpy as jnp
from jax import lax
from jax.experimental import pallas as pl
from jax.experimental.pallas import tpu as pltpu
from jax.experimental.pallas import tpu_sc as plsc

N_NEIGHBORS = 16
N_CLASSES = 10
TILE = 1024
CHUNK = 8
NCH = TILE // CHUNK
QB3 = 128
PAD_VAL = 1.0e9


def _extract_topk_with_labels(vals, labs, k, want_vals=True):
    B, W = vals.shape
    iota = lax.broadcasted_iota(jnp.int32, (B, W), 1)
    out_v, out_l = [], []
    for _ in range(k):
        m = jnp.min(vals, axis=1, keepdims=True)
        am = jnp.min(jnp.where(vals == m, iota, W), axis=1, keepdims=True)
        sel = iota == am
        lab = jnp.min(jnp.where(sel, labs, jnp.int32(2**30)), axis=1,
                      keepdims=True)
        vals = jnp.where(sel, jnp.float32(jnp.inf), vals)
        out_v.append(m)
        out_l.append(lab)
    vcat = jnp.concatenate(out_v, axis=1) if want_vals else None
    return vcat, jnp.concatenate(out_l, axis=1)


def _vote(run_v, run_l):
    B = run_v.shape[0]
    d = jnp.sqrt(jnp.maximum(run_v, 0.0))
    dinv = 1.0 / d
    inf_mask = jnp.isinf(dinv)
    inf_row = jnp.max(inf_mask.astype(jnp.float32), axis=1, keepdims=True) > 0
    w = jnp.where(inf_row, inf_mask.astype(jnp.float32), dinv)
    votes = [
        jnp.sum(jnp.where(run_l == c, w, 0.0), axis=1, keepdims=True)
        for c in range(N_CLASSES)
    ]
    proba = jnp.concatenate(votes, axis=1)
    s = jnp.sum(proba, axis=1, keepdims=True)
    s = jnp.where(s == 0.0, jnp.float32(1.0), s)
    proba = proba * (1.0 / s)
    pm = jnp.max(proba, axis=1, keepdims=True)
    ci = lax.broadcasted_iota(jnp.int32, (B, N_CLASSES), 1)
    pred = jnp.min(jnp.where(proba == pm, ci, jnp.int32(N_CLASSES)), axis=1,
                   keepdims=True)
    return pred, proba


def _stage_a_kernel(x_ref, t_ref, m_ref, tm_ref):
    x = x_ref[...]
    t = t_ref[0]
    dn = (((1,), (1,)), ((), ()))
    xm2 = -2.0 * x
    xym2 = lax.dot_general(xm2, t, dn, preferred_element_type=jnp.float32)
    ones = jnp.ones((8, x.shape[1]), jnp.float32)
    t2r = lax.dot_general(ones, t * t, dn,
                          preferred_element_type=jnp.float32,
                          precision=lax.Precision.HIGHEST)[0:1]
    s = t2r + xym2
    f1 = jnp.minimum(s[:, :512], s[:, 512:])
    f2 = jnp.minimum(f1[:, :256], f1[:, 256:])
    mt = jnp.minimum(f2[:, :128], f2[:, 128:])
    m_ref[0] = mt
    tm_ref[0] = jnp.min(mt, axis=1, keepdims=True)


def _stage_a(x, tr3, n_tiles):
    B, F = x.shape
    return pl.pallas_call(
        _stage_a_kernel,
        grid=(n_tiles,),
        in_specs=[
            pl.BlockSpec((B, F), lambda i: (0, 0)),
            pl.BlockSpec((1, TILE, F), lambda i: (i, 0, 0)),
        ],
        out_specs=[
            pl.BlockSpec((1, B, NCH), lambda i: (i, 0, 0)),
            pl.BlockSpec((1, B, 1), lambda i: (i, 0, 0)),
        ],
        out_shape=[
            jax.ShapeDtypeStruct((n_tiles, B, NCH), jnp.float32),
            jax.ShapeDtypeStruct((n_tiles, B, 1), jnp.float32),
        ],
    )(x, tr3)


def _stage_b_kernel(b_total, m2_ref, gidx_ref, sig_ref):
    m2t = m2_ref[...].T
    B, n_tiles = m2t.shape
    iota = lax.broadcasted_iota(jnp.int32, (B, n_tiles), 1)
    _, sig = _extract_topk_with_labels(m2t, iota, N_NEIGHBORS, want_vals=False)
    qidx = lax.broadcasted_iota(jnp.int32, (B, 1), 0)
    gidx_ref[...] = sig * b_total + qidx
    sig_ref[...] = sig


def _stage_b(m2):
    n_tiles, B = m2.shape
    return pl.pallas_call(
        functools.partial(_stage_b_kernel, B),
        in_specs=[pl.BlockSpec((n_tiles, B), lambda: (0, 0))],
        out_specs=[
            pl.BlockSpec((B, N_NEIGHBORS), lambda: (0, 0)),
            pl.BlockSpec((B, N_NEIGHBORS), lambda: (0, 0)),
        ],
        out_shape=[
            jax.ShapeDtypeStruct((B, N_NEIGHBORS), jnp.int32),
            jax.ShapeDtypeStruct((B, N_NEIGHBORS), jnp.int32),
        ],
    )(m2)


def _stage_c2_kernel(n_real, cand_ref, sig_ref, raw_ref, clamp_ref):
    sig = sig_ref[...]
    i128 = lax.broadcasted_iota(jnp.int32, (1, NCH), 1)
    kaps = []
    for k in range(N_NEIGHBORS):
        kaps.append(sig[:, k:k + 1] * NCH + i128)
    kappa = jnp.concatenate(kaps, axis=1)
    _, kap = _extract_topk_with_labels(cand_ref[...], kappa,
                                       N_NEIGHBORS, want_vals=False)
    i8 = lax.broadcasted_iota(jnp.int32, (1, CHUNK), 1)
    rows = []
    for k in range(N_NEIGHBORS):
        kk = kap[:, k:k + 1]
        rows.append((kk >> 7) * TILE + (kk & 127) + 128 * i8)
    raw = jnp.concatenate(rows, axis=1)
    raw_ref[...] = raw
    clamp_ref[...] = jnp.minimum(raw, jnp.int32(n_real - 1))


def _stage_c2(cand, sig, n_real):
    B = sig.shape[0]
    W = N_NEIGHBORS * CHUNK
    return pl.pallas_call(
        functools.partial(_stage_c2_kernel, n_real),
        in_specs=[
            pl.BlockSpec((B, N_NEIGHBORS * NCH), lambda: (0, 0)),
            pl.BlockSpec((B, N_NEIGHBORS), lambda: (0, 0)),
        ],
        out_specs=[
            pl.BlockSpec((B, W), lambda: (0, 0)),
            pl.BlockSpec((B, W), lambda: (0, 0)),
        ],
        out_shape=[
            jax.ShapeDtypeStruct((B, W), jnp.int32),
            jax.ShapeDtypeStruct((B, W), jnp.int32),
        ],
    )(cand, sig)


def _stage_c3_kernel(n_real, n_blocks, x_ref, g_ref, raw_ref, pred_ref,
                     proba_ref, d2s_ref, lab_s_ref):
    gi = pl.program_id(0)
    x = x_ref[...]
    g = g_ref[...]
    raw = raw_ref[...]
    gt = g[:, :, :64]
    x2 = jnp.sum(x * x, axis=1, keepdims=True)
    xb = x.astype(jnp.bfloat16).astype(jnp.float32)
    gb = gt.astype(jnp.bfloat16).astype(jnp.float32)
    xg = jnp.sum(gb * xb[:, None, :], axis=2)
    g2 = jnp.sum(gt * gt, axis=2)
    d2 = (x2 + g2) - 2.0 * xg
    d2 = jnp.where(raw >= n_real, jnp.float32(jnp.inf), d2)
    d2s_ref[pl.ds(gi * QB3, QB3), :] = d2
    lab_s_ref[pl.ds(gi * QB3, QB3), :] = g[:, :, 64].astype(jnp.int32)

    @pl.when(gi == n_blocks - 1)
    def _finish():
        vals, labs = _extract_topk_with_labels(d2s_ref[...], lab_s_ref[...],
                                               N_NEIGHBORS)
        pred, proba = _vote(vals, labs)
        pred_ref[...] = pred
        proba_ref[...] = proba


def _stage_c3(x, gath, raw, n_real):
    B, F = x.shape
    W = N_NEIGHBORS * CHUNK
    n_blocks = B // QB3
    return pl.pallas_call(
        functools.partial(_stage_c3_kernel, n_real, n_blocks),
        grid=(n_blocks,),
        in_specs=[
            pl.BlockSpec((QB3, F), lambda g: (g, 0)),
            pl.BlockSpec((QB3, W, 128), lambda g: (g, 0, 0)),
            pl.BlockSpec((QB3, W), lambda g: (g, 0)),
        ],
        out_specs=[
            pl.BlockSpec((B, 1), lambda g: (0, 0)),
            pl.BlockSpec((B, N_CLASSES), lambda g: (0, 0)),
        ],
        out_shape=[
            jax.ShapeDtypeStruct((B, 1), jnp.int32),
            jax.ShapeDtypeStruct((B, N_CLASSES), jnp.float32),
        ],
        scratch_shapes=[
            pltpu.VMEM((B, W), jnp.float32),
            pltpu.VMEM((B, W), jnp.int32),
        ],
    )(x, gath, raw)


def _sc_gather(data, idx_flat, window=128):
    K = idx_flat.shape[0]
    D = data.shape[1]
    idx2 = idx_flat.reshape(1, K)
    mesh = plsc.VectorSubcoreMesh(core_axis_name="c", subcore_axis_name="s")
    cp = pltpu.CompilerParams()
    if "needs_layout_passes" in pltpu.CompilerParams.__dataclass_fields__:
        cp = dataclasses.replace(cp, needs_layout_passes=False)

    @functools.partial(
        pl.kernel,
        out_type=jax.ShapeDtypeStruct((K, D), data.dtype),
        mesh=mesh,
        compiler_params=cp,
    )
    def gk(x_hbm, i_hbm, o_hbm):
        def body(i_vmem, o_vmem):
            pltpu.sync_copy(x_hbm.at[i_vmem.at[0]], o_vmem)

        pltpu.emit_pipeline(
            body,
            grid=(K // window,),
            in_specs=[pl.BlockSpec((1, window), lambda i: (0, i))],
            out_specs=[pl.BlockSpec((window, D), lambda i: (i, 0))],
            core_axis_name=("c", "s"),
            dimension_semantics=(pltpu.PARALLEL,),
        )(i_hbm, o_hbm)

    return gk(data, idx2)


def kernel(x, train_data, train_labels):
    B, F = x.shape
    N = train_data.shape[0]
    n_tiles = (N + TILE - 1) // TILE
    n_pad = n_tiles * TILE
    tr3 = jnp.pad(train_data, ((0, n_pad - N), (0, 0)),
                  constant_values=PAD_VAL).reshape(n_tiles, TILE, F)
    aug = jnp.concatenate(
        [train_data, train_labels.astype(jnp.float32)[:, None],
         jnp.zeros((N, 63), jnp.float32)], axis=1)

    m3, tm = _stage_a(x, tr3, n_tiles)
    gidx, sig = _stage_b(tm.reshape(n_tiles, B))
    g1 = _sc_gather(m3.reshape(n_tiles * B, NCH),
                    gidx.reshape(B * N_NEIGHBORS))
    cand = g1.reshape(B, N_NEIGHBORS * NCH)
    raw, clamped = _stage_c2(cand, sig, N)
    g2 = _sc_gather(aug, clamped.reshape(B * N_NEIGHBORS * CHUNK))
    gath = g2.reshape(B, N_NEIGHBORS * CHUNK, 128)
    pred2, proba = _stage_c3(x, gath, raw, N)
    return (pred2.reshape(B), proba)


# speedup vs baseline: 1.0764x; 1.0132x over previous
import dataclasses
import functools

import jax
import jax.numpy as jnp
from jax import lax
from jax.experimental import pallas as pl
from jax.experimental.pallas import tpu as pltpu
from jax.experimental.pallas import tpu_sc as plsc

N_NEIGHBORS = 16
N_CLASSES = 10
TILE = 1024
CHUNK = 8
NCH = TILE // CHUNK
QB3 = 128
PAD_VAL = 1.0e9


def _extract_topk_with_labels(vals, labs, k, want_vals=True):
    B, W = vals.shape
    iota = lax.broadcasted_iota(jnp.int32, (B, W), 1)
    out_v, out_l = [], []
    for _ in range(k):
        m = jnp.min(vals, axis=1, keepdims=True)
        am = jnp.min(jnp.where(vals == m, iota, W), axis=1, keepdims=True)
        sel = iota == am
        lab = jnp.min(jnp.where(sel, labs, jnp.int32(2**30)), axis=1,
                      keepdims=True)
        vals = jnp.where(sel, jnp.float32(jnp.inf), vals)
        out_v.append(m)
        out_l.append(lab)
    vcat = jnp.concatenate(out_v, axis=1) if want_vals else None
    return vcat, jnp.concatenate(out_l, axis=1)


def _vote(run_v, run_l):
    B = run_v.shape[0]
    d = jnp.sqrt(jnp.maximum(run_v, 0.0))
    dinv = 1.0 / d
    inf_mask = jnp.isinf(dinv)
    inf_row = jnp.max(inf_mask.astype(jnp.float32), axis=1, keepdims=True) > 0
    w = jnp.where(inf_row, inf_mask.astype(jnp.float32), dinv)
    votes = [
        jnp.sum(jnp.where(run_l == c, w, 0.0), axis=1, keepdims=True)
        for c in range(N_CLASSES)
    ]
    proba = jnp.concatenate(votes, axis=1)
    s = jnp.sum(proba, axis=1, keepdims=True)
    s = jnp.where(s == 0.0, jnp.float32(1.0), s)
    proba = proba * (1.0 / s)
    pm = jnp.max(proba, axis=1, keepdims=True)
    ci = lax.broadcasted_iota(jnp.int32, (B, N_CLASSES), 1)
    pred = jnp.min(jnp.where(proba == pm, ci, jnp.int32(N_CLASSES)), axis=1,
                   keepdims=True)
    return pred, proba


def _stage_a_kernel(x_ref, t_ref, m_ref, tm_ref):
    x = x_ref[...]
    t = t_ref[0]
    dn = (((1,), (1,)), ((), ()))
    xy = lax.dot_general(x, t, dn, preferred_element_type=jnp.float32)
    ones = jnp.ones((8, x.shape[1]), jnp.float32)
    t2r = lax.dot_general(ones, t * t, dn,
                          preferred_element_type=jnp.float32,
                          precision=lax.Precision.HIGHEST)[0:1]
    s = t2r - 2.0 * xy
    f1 = jnp.minimum(s[:, :512], s[:, 512:])
    f2 = jnp.minimum(f1[:, :256], f1[:, 256:])
    mt = jnp.minimum(f2[:, :128], f2[:, 128:])
    m_ref[0] = mt
    tm_ref[0] = jnp.min(mt, axis=1, keepdims=True)


def _stage_a(x, tr3, n_tiles):
    B, F = x.shape
    return pl.pallas_call(
        _stage_a_kernel,
        grid=(n_tiles,),
        in_specs=[
            pl.BlockSpec((B, F), lambda i: (0, 0)),
            pl.BlockSpec((1, TILE, F), lambda i: (i, 0, 0)),
        ],
        out_specs=[
            pl.BlockSpec((1, B, NCH), lambda i: (i, 0, 0)),
            pl.BlockSpec((1, B, 1), lambda i: (i, 0, 0)),
        ],
        out_shape=[
            jax.ShapeDtypeStruct((n_tiles, B, NCH), jnp.float32),
            jax.ShapeDtypeStruct((n_tiles, B, 1), jnp.float32),
        ],
    )(x, tr3)


def _stage_b_kernel(b_total, m2_ref, gidx_ref, sig_ref):
    m2t = m2_ref[...].T
    B, n_tiles = m2t.shape
    iota = lax.broadcasted_iota(jnp.int32, (B, n_tiles), 1)
    _, sig = _extract_topk_with_labels(m2t, iota, N_NEIGHBORS, want_vals=False)
    qidx = lax.broadcasted_iota(jnp.int32, (B, 1), 0)
    gidx_ref[...] = sig * b_total + qidx
    sig_ref[...] = sig


def _stage_b(m2):
    n_tiles, B = m2.shape
    return pl.pallas_call(
        functools.partial(_stage_b_kernel, B),
        in_specs=[pl.BlockSpec((n_tiles, B), lambda: (0, 0))],
        out_specs=[
            pl.BlockSpec((B, N_NEIGHBORS), lambda: (0, 0)),
            pl.BlockSpec((B, N_NEIGHBORS), lambda: (0, 0)),
        ],
        out_shape=[
            jax.ShapeDtypeStruct((B, N_NEIGHBORS), jnp.int32),
            jax.ShapeDtypeStruct((B, N_NEIGHBORS), jnp.int32),
        ],
    )(m2)


def _stage_c2_kernel(n_real, cand_ref, sig_ref, raw_ref, clamp_ref):
    sig = sig_ref[...]
    i128 = lax.broadcasted_iota(jnp.int32, (1, NCH), 1)
    kaps = []
    for k in range(N_NEIGHBORS):
        kaps.append(sig[:, k:k + 1] * NCH + i128)
    kappa = jnp.concatenate(kaps, axis=1)
    _, kap = _extract_topk_with_labels(cand_ref[...], kappa,
                                       N_NEIGHBORS, want_vals=False)
    i8 = lax.broadcasted_iota(jnp.int32, (1, CHUNK), 1)
    rows = []
    for k in range(N_NEIGHBORS):
        kk = kap[:, k:k + 1]
        rows.append((kk >> 7) * TILE + (kk & 127) + 128 * i8)
    raw = jnp.concatenate(rows, axis=1)
    raw_ref[...] = raw
    clamp_ref[...] = jnp.minimum(raw, jnp.int32(n_real - 1))


def _stage_c2(cand, sig, n_real):
    B = sig.shape[0]
    W = N_NEIGHBORS * CHUNK
    return pl.pallas_call(
        functools.partial(_stage_c2_kernel, n_real),
        in_specs=[
            pl.BlockSpec((B, N_NEIGHBORS * NCH), lambda: (0, 0)),
            pl.BlockSpec((B, N_NEIGHBORS), lambda: (0, 0)),
        ],
        out_specs=[
            pl.BlockSpec((B, W), lambda: (0, 0)),
            pl.BlockSpec((B, W), lambda: (0, 0)),
        ],
        out_shape=[
            jax.ShapeDtypeStruct((B, W), jnp.int32),
            jax.ShapeDtypeStruct((B, W), jnp.int32),
        ],
    )(cand, sig)


def _stage_c3_kernel(n_real, n_blocks, x_ref, g_ref, raw_ref, pred_ref,
                     proba_ref, d2s_ref, lab_s_ref):
    gi = pl.program_id(0)
    x = x_ref[...]
    g = g_ref[...]
    raw = raw_ref[...]
    gt = g[:, :, :64]
    x2 = jnp.sum(x * x, axis=1, keepdims=True)
    xb = x.astype(jnp.bfloat16).astype(jnp.float32)
    gb = gt.astype(jnp.bfloat16).astype(jnp.float32)
    xg = jnp.sum(gb * xb[:, None, :], axis=2)
    g2 = jnp.sum(gt * gt, axis=2)
    d2 = (x2 + g2) - 2.0 * xg
    d2 = jnp.where(raw >= n_real, jnp.float32(jnp.inf), d2)
    d2s_ref[pl.ds(gi * QB3, QB3), :] = d2
    lab_s_ref[pl.ds(gi * QB3, QB3), :] = g[:, :, 64].astype(jnp.int32)

    @pl.when(gi == n_blocks - 1)
    def _finish():
        vals, labs = _extract_topk_with_labels(d2s_ref[...], lab_s_ref[...],
                                               N_NEIGHBORS)
        pred, proba = _vote(vals, labs)
        pred_ref[...] = pred
        proba_ref[...] = proba


def _stage_c3(x, gath, raw, n_real):
    B, F = x.shape
    W = N_NEIGHBORS * CHUNK
    n_blocks = B // QB3
    return pl.pallas_call(
        functools.partial(_stage_c3_kernel, n_real, n_blocks),
        grid=(n_blocks,),
        in_specs=[
            pl.BlockSpec((QB3, F), lambda g: (g, 0)),
            pl.BlockSpec((QB3, W, 128), lambda g: (g, 0, 0)),
            pl.BlockSpec((QB3, W), lambda g: (g, 0)),
        ],
        out_specs=[
            pl.BlockSpec((B, 1), lambda g: (0, 0)),
            pl.BlockSpec((B, N_CLASSES), lambda g: (0, 0)),
        ],
        out_shape=[
            jax.ShapeDtypeStruct((B, 1), jnp.int32),
            jax.ShapeDtypeStruct((B, N_CLASSES), jnp.float32),
        ],
        scratch_shapes=[
            pltpu.VMEM((B, W), jnp.float32),
            pltpu.VMEM((B, W), jnp.int32),
        ],
    )(x, gath, raw)


def _sc_gather(data, idx_flat, window=128):
    K = idx_flat.shape[0]
    D = data.shape[1]
    idx2 = idx_flat.reshape(1, K)
    mesh = plsc.VectorSubcoreMesh(core_axis_name="c", subcore_axis_name="s")
    cp = pltpu.CompilerParams()
    if "needs_layout_passes" in pltpu.CompilerParams.__dataclass_fields__:
        cp = dataclasses.replace(cp, needs_layout_passes=False)

    @functools.partial(
        pl.kernel,
        out_type=jax.ShapeDtypeStruct((K, D), data.dtype),
        mesh=mesh,
        compiler_params=cp,
    )
    def gk(x_hbm, i_hbm, o_hbm):
        def body(i_vmem, o_vmem):
            pltpu.sync_copy(x_hbm.at[i_vmem.at[0]], o_vmem)

        pltpu.emit_pipeline(
            body,
            grid=(K // window,),
            in_specs=[pl.BlockSpec((1, window), lambda i: (0, i))],
            out_specs=[pl.BlockSpec((window, D), lambda i: (i, 0))],
            core_axis_name=("c", "s"),
            dimension_semantics=(pltpu.PARALLEL,),
        )(i_hbm, o_hbm)

    return gk(data, idx2)


def kernel(x, train_data, train_labels):
    B, F = x.shape
    N = train_data.shape[0]
    n_tiles = (N + TILE - 1) // TILE
    n_pad = n_tiles * TILE
    tr3 = jnp.pad(train_data, ((0, n_pad - N), (0, 0)),
                  constant_values=PAD_VAL).reshape(n_tiles, TILE, F)
    aug = jnp.concatenate(
        [train_data, train_labels.astype(jnp.float32)[:, None],
         jnp.zeros((N, 63), jnp.float32)], axis=1)

    m3, tm = _stage_a(x, tr3, n_tiles)
    gidx, sig = _stage_b(tm.reshape(n_tiles, B))
    g1 = _sc_gather(m3.reshape(n_tiles * B, NCH),
                    gidx.reshape(B * N_NEIGHBORS))
    cand = g1.reshape(B, N_NEIGHBORS * NCH)
    raw, clamped = _stage_c2(cand, sig, N)
    g2 = _sc_gather(aug, clamped.reshape(B * N_NEIGHBORS * CHUNK))
    gath = g2.reshape(B, N_NEIGHBORS * CHUNK, 128)
    pred2, proba = _stage_c3(x, gath, raw, N)
    return (pred2.reshape(B), proba)
